# baseline (speedup 1.0000x reference)
import numpy as np
import jax
import jax.numpy as jnp
from jax.experimental import pallas as pl
from jax.experimental.pallas import tpu as pltpu


def _build_patches(xpad, S):
    N, Hp, Wp, C = xpad.shape
    Ho = (Hp - 2) // 2
    G = (Wp - 2) // (2 * S)
    rs = []
    for r in range(4):
        xr = xpad[:, r:r + 2 * Ho:2]
        a = xr[:, :, 1:1 + 2 * S * G].reshape(N, Ho, G, 2 * S, C)
        b0 = xr[:, :, 0:2 * S * G:2 * S][:, :, :, None, :]
        b1 = xr[:, :, 2 * S + 1::2 * S][:, :, :, None, :]
        rs.append(jnp.concatenate([b0, a, b1], axis=3))
    pf = jnp.stack(rs, axis=3)
    return pf.reshape(N * Ho * G, 4 * (2 * S + 2) * C)


def _tap_weights(w_mat, S, Cin, Cout):
    Kp = 4 * (2 * S + 2) * Cin
    src = np.arange(9 * Cin)
    kh = src // (3 * Cin)
    kw = (src // Cin) % 3
    ci = src % Cin
    mats = []
    for dh in range(2):
        for dw in range(2):
            wt = jnp.zeros((Kp, S * Cout), w_mat.dtype)
            for s in range(S):
                dst = ((dh + kh) * (2 * S + 2) + (2 * s + dw + kw)) * Cin + ci
                wt = wt.at[dst, s * Cout:(s + 1) * Cout].set(w_mat)
            mats.append(wt.astype(jnp.bfloat16))
    return mats


def _conv_body(p_ref, w0_ref, w1_ref, w2_ref, w3_ref, b_ref, o_ref):
    p = p_ref[...]
    z = jnp.dot(p, w0_ref[...], preferred_element_type=jnp.float32)
    z = jnp.maximum(z, jnp.dot(p, w1_ref[...], preferred_element_type=jnp.float32))
    z = jnp.maximum(z, jnp.dot(p, w2_ref[...], preferred_element_type=jnp.float32))
    z = jnp.maximum(z, jnp.dot(p, w3_ref[...], preferred_element_type=jnp.float32))
    o_ref[...] = jnp.maximum(z + b_ref[...], 0.0).astype(o_ref.dtype)


def _conv_stage(pf, wmats, bias, S, Cout, tm):
    M, Kp = pf.shape
    assert M % tm == 0
    btile = jnp.tile(bias, S).reshape(1, S * Cout).astype(jnp.float32)
    return pl.pallas_call(
        _conv_body,
        out_shape=jax.ShapeDtypeStruct((M, S * Cout), jnp.bfloat16),
        grid=(M // tm,),
        in_specs=[pl.BlockSpec((tm, Kp), lambda i: (i, 0))] + [
            pl.BlockSpec((Kp, S * Cout), lambda i: (0, 0))] * 4 + [
            pl.BlockSpec((1, S * Cout), lambda i: (0, 0)),
        ],
        out_specs=pl.BlockSpec((tm, S * Cout), lambda i: (i, 0)),
        compiler_params=pltpu.CompilerParams(
            dimension_semantics=("parallel",)),
    )(pf, *wmats, btile)


def _mlp_body(x_ref, w1_ref, b1_ref, w2_ref, b2_ref, o_ref, acc_ref):
    k = pl.program_id(0)

    @pl.when(k == 0)
    def _():
        acc_ref[...] = jnp.zeros_like(acc_ref)

    acc_ref[...] += jnp.dot(x_ref[...].astype(jnp.float32), w1_ref[...],
                            preferred_element_type=jnp.float32)

    @pl.when(k == pl.num_programs(0) - 1)
    def _():
        h = jnp.maximum(acc_ref[...] + b1_ref[...], 0.0)
        o_ref[...] = (jnp.dot(h, w2_ref[...],
                              preferred_element_type=jnp.float32)
                      + b2_ref[...])


def _mlp_head(x, w1, b1, w2, b2, tk):
    N, K = x.shape
    Hd = w1.shape[1]
    O = w2.shape[1]
    assert K % tk == 0
    return pl.pallas_call(
        _mlp_body,
        out_shape=jax.ShapeDtypeStruct((N, O), jnp.float32),
        grid=(K // tk,),
        in_specs=[
            pl.BlockSpec((N, tk), lambda k: (0, k)),
            pl.BlockSpec((tk, Hd), lambda k: (k, 0)),
            pl.BlockSpec((1, Hd), lambda k: (0, 0)),
            pl.BlockSpec((Hd, O), lambda k: (0, 0)),
            pl.BlockSpec((1, O), lambda k: (0, 0)),
        ],
        out_specs=pl.BlockSpec((N, O), lambda k: (0, 0)),
        scratch_shapes=[pltpu.VMEM((N, Hd), jnp.float32)],
        compiler_params=pltpu.CompilerParams(
            dimension_semantics=("arbitrary",),
            vmem_limit_bytes=64 * 1024 * 1024,
        ),
    )(x, w1, b1.reshape(1, Hd), w2, b2.reshape(1, O))


def kernel(x_nchw, conv1_w, conv1_b, conv2_w, conv2_b, fc1_w, fc1_b,
           fc2_w, fc2_b):
    N = x_nchw.shape[0]

    x = jnp.transpose(x_nchw, (0, 2, 3, 1)).astype(jnp.bfloat16)
    xp = jnp.pad(x, ((0, 0), (1, 1), (1, 1), (0, 0)))

    pf1 = _build_patches(xp, S=8)
    w1mats = _tap_weights(conv1_w, S=8, Cin=3, Cout=16)
    y1 = _conv_stage(pf1, w1mats, conv1_b, S=8, Cout=16, tm=pf1.shape[0] // 14)
    y1 = y1.reshape(N, 112, 112, 16)

    yp = jnp.pad(y1, ((0, 0), (1, 1), (1, 1), (0, 0)))
    pf2 = _build_patches(yp, S=4)
    w2mats = _tap_weights(conv2_w, S=4, Cin=16, Cout=32)
    y2 = _conv_stage(pf2, w2mats, conv2_b, S=4, Cout=32, tm=pf2.shape[0] // 14)

    flat = y2.reshape(N, 56 * 56 * 32)
    return _mlp_head(flat, fc1_w, fc1_b, fc2_w, fc2_b, tk=7168)


# speedup vs baseline: 2.4330x; 2.4330x over previous
import numpy as np
import jax
import jax.numpy as jnp
from jax.experimental import pallas as pl
from jax.experimental.pallas import tpu as pltpu


def _tap_weights(w_mat, S, Cin, Cout, ci_major):
    W = 2 * S + 2
    Kp = 4 * W * Cin
    src = np.arange(9 * Cin)
    kh = src // (3 * Cin)
    kw = (src // Cin) % 3
    ci = src % Cin
    mats = []
    for dh in range(2):
        for dw in range(2):
            wt = jnp.zeros((Kp, S * Cout), w_mat.dtype)
            for s in range(S):
                r = dh + kh
                c = 2 * s + dw + kw
                if ci_major:
                    dst = (r * Cin + ci) * W + c
                else:
                    dst = (r * W + c) * Cin + ci
                wt = wt.at[dst, s * Cout:(s + 1) * Cout].set(w_mat)
            mats.append(wt.astype(jnp.bfloat16))
    return mats


def _tap_dots_epilogue(pf_ref, w_refs, b_ref, o_ref):
    p = pf_ref[...]
    z = jnp.dot(p, w_refs[0][...], preferred_element_type=jnp.float32)
    for wr in w_refs[1:]:
        z = jnp.maximum(z, jnp.dot(p, wr[...],
                                   preferred_element_type=jnp.float32))
    o_ref[...] = jnp.maximum(z + b_ref[...], 0.0).astype(o_ref.dtype)


_PHASE1 = {(ph, r): ((2 * ph + r - 1) % 4, (2 * ph + r - 1 - (2 * ph + r - 1) % 4) // 4)
           for ph in range(2) for r in range(4)}
_PHASE2 = {r: ((r - 1) % 2, (r - 1 - (r - 1) % 2) // 2) for r in range(4)}


def _conv1_body(x_ref, w0, w1, w2, w3, b_ref, o_ref, pf_ref):
    IB = x_ref.shape[0]
    pf_ref[...] = jnp.zeros_like(pf_ref)
    for img in range(IB):
        planes = {}
        for ci in range(3):
            pb = x_ref[img, ci].astype(jnp.bfloat16)
            pbm = jnp.concatenate(
                [pb, jnp.zeros((224, 32), jnp.bfloat16)], axis=1)
            pbm = pbm.reshape(56, 1024)
            for pp in range(4):
                planes[(ci, pp)] = pbm[:, pp * 256:pp * 256 + 224]
        for g in range(14):
            cs = 16 * g - 1
            src_lo, src_hi = max(cs, 0), min(cs + 18, 224)
            dst_lo = src_lo - cs
            for ph in range(2):
                for r in range(4):
                    pp, dlt = _PHASE1[(ph, r)]
                    if dlt == 0:
                        qs, qe, ds, de = 0, 56, 0, 56
                    elif dlt < 0:
                        qs, qe, ds, de = 0, 55, 1, 56
                    else:
                        qs, qe, ds, de = 1, 56, 0, 55
                    rb = img * 1568 + g * 112 + ph * 56
                    for ci in range(3):
                        lane = (r * 3 + ci) * 18 + dst_lo
                        pf_ref[rb + ds:rb + de, lane:lane + src_hi - src_lo] = \
                            planes[(ci, pp)][qs:qe, src_lo:src_hi]
    _tap_dots_epilogue(pf_ref, (w0, w1, w2, w3), b_ref, o_ref)


def _conv2_body(y_ref, w0, w1, w2, w3, b_ref, o_ref, pf_ref):
    IB = y_ref.shape[0] // 1568
    pf_ref[...] = jnp.zeros_like(pf_ref)
    for img in range(IB):
        for g2 in range(14):
            for r in range(4):
                ph, dlt = _PHASE2[r]
                if dlt == 0:
                    qs, qe, ds, de = 0, 56, 0, 56
                elif dlt < 0:
                    qs, qe, ds, de = 0, 55, 1, 56
                else:
                    qs, qe, ds, de = 1, 56, 0, 55
                rb = img * 784 + g2 * 56
                lane0 = r * 160
                pieces = []
                if g2 > 0:
                    pieces.append((g2 - 1, 112, 16, lane0))
                pieces.append((g2, 0, 128, lane0 + 16))
                if g2 < 13:
                    pieces.append((g2 + 1, 0, 16, lane0 + 144))
                for (gs, ls, lw, dl) in pieces:
                    sb = img * 1568 + gs * 112 + ph * 56
                    pf_ref[rb + ds:rb + de, dl:dl + lw] = \
                        y_ref[sb + qs:sb + qe, ls:ls + lw]
    _tap_dots_epilogue(pf_ref, (w0, w1, w2, w3), b_ref, o_ref)


def _conv_stage(body, x, xblk, wmats, bias, S, Cout, mrows, ib, kp):
    n_img = x.shape[0] if body is _conv1_body else x.shape[0] // 1568
    grid = (n_img + ib - 1) // ib
    btile = jnp.tile(bias, S).reshape(1, S * Cout).astype(jnp.float32)
    return pl.pallas_call(
        body,
        out_shape=jax.ShapeDtypeStruct((n_img * mrows, S * Cout), jnp.bfloat16),
        grid=(grid,),
        in_specs=[pl.BlockSpec(xblk, lambda i: (i,) + (0,) * (len(xblk) - 1))] + [
            pl.BlockSpec((kp, S * Cout), lambda i: (0, 0))] * 4 + [
            pl.BlockSpec((1, S * Cout), lambda i: (0, 0)),
        ],
        out_specs=pl.BlockSpec((ib * mrows, S * Cout), lambda i: (i, 0)),
        scratch_shapes=[pltpu.VMEM((ib * mrows, kp), jnp.bfloat16)],
        compiler_params=pltpu.CompilerParams(
            dimension_semantics=("parallel",)),
    )(x, *wmats, btile)


def _mlp_body(x_ref, w1_ref, b1_ref, w2_ref, b2_ref, o_ref, acc_ref):
    k = pl.program_id(0)

    @pl.when(k == 0)
    def _():
        acc_ref[...] = jnp.zeros_like(acc_ref)

    w1t = w1_ref[...].reshape(-1, w1_ref.shape[-1])
    acc_ref[...] += jnp.dot(x_ref[...].astype(jnp.float32), w1t,
                            preferred_element_type=jnp.float32)

    @pl.when(k == pl.num_programs(0) - 1)
    def _():
        h = jnp.maximum(acc_ref[...] + b1_ref[...], 0.0)
        o_ref[...] = (jnp.dot(h, w2_ref[...],
                              preferred_element_type=jnp.float32)
                      + b2_ref[...])


def _mlp_head(x, w1v, b1, w2, b2):
    N = x.shape[0]
    Hd = w1v.shape[-1]
    O = w2.shape[1]
    tk = 56 * 4 * 32
    return pl.pallas_call(
        _mlp_body,
        out_shape=jax.ShapeDtypeStruct((N, O), jnp.float32),
        grid=(14,),
        in_specs=[
            pl.BlockSpec((N, tk), lambda k: (0, k)),
            pl.BlockSpec((56, 1, 4, 32, Hd), lambda k: (0, k, 0, 0, 0)),
            pl.BlockSpec((1, Hd), lambda k: (0, 0)),
            pl.BlockSpec((Hd, O), lambda k: (0, 0)),
            pl.BlockSpec((1, O), lambda k: (0, 0)),
        ],
        out_specs=pl.BlockSpec((N, O), lambda k: (0, 0)),
        scratch_shapes=[pltpu.VMEM((N, Hd), jnp.float32)],
        compiler_params=pltpu.CompilerParams(
            dimension_semantics=("arbitrary",),
            vmem_limit_bytes=64 * 1024 * 1024,
        ),
    )(x, w1v, b1.reshape(1, Hd), w2, b2.reshape(1, O))


def kernel(x_nchw, conv1_w, conv1_b, conv2_w, conv2_b, fc1_w, fc1_b,
           fc2_w, fc2_b):
    N = x_nchw.shape[0]
    w1mats = _tap_weights(conv1_w, S=8, Cin=3, Cout=16, ci_major=True)
    w2mats = _tap_weights(conv2_w, S=4, Cin=16, Cout=32, ci_major=False)

    y1 = _conv_stage(_conv1_body, x_nchw, (2, 3, 224, 224), w1mats, conv1_b,
                     S=8, Cout=16, mrows=1568, ib=2, kp=216)
    y2 = _conv_stage(_conv2_body, y1, (2 * 1568, 128), w2mats, conv2_b,
                     S=4, Cout=32, mrows=784, ib=2, kp=640)

    flat = y2.reshape(N, 56 * 56 * 32)
    w1v = fc1_w.reshape(56, 14, 4, 32, 128)
    return _mlp_head(flat, w1v, fc1_b, fc2_w, fc2_b)


# speedup vs baseline: 124.3234x; 51.0978x over previous
import numpy as np
import jax
import jax.numpy as jnp
from jax.experimental import pallas as pl
from jax.experimental.pallas import tpu as pltpu


def _tap_weights(w_mat, S, Cin, Cout, ci_major):
    W = 2 * S + 2
    Kp = 4 * W * Cin
    P = np.zeros((4, Kp, S * 9 * Cin), np.float32)
    for t, (dh, dw) in enumerate([(0, 0), (0, 1), (1, 0), (1, 1)]):
        for s in range(S):
            for kh in range(3):
                for kw in range(3):
                    for ci in range(Cin):
                        r = dh + kh
                        c = 2 * s + dw + kw
                        if ci_major:
                            dst = (r * Cin + ci) * W + c
                        else:
                            dst = (r * W + c) * Cin + ci
                        src = (kh * 3 + kw) * Cin + ci
                        P[t, dst, s * 9 * Cin + src] = 1.0
    wrep = (jnp.eye(S, dtype=w_mat.dtype)[:, None, :, None]
            * w_mat[None, :, None, :]).reshape(S * 9 * Cin, S * Cout)
    wall = jnp.einsum('tkm,mn->tkn', jnp.asarray(P), wrep).astype(jnp.bfloat16)
    return [wall[t] for t in range(4)]


def _tap_dots_epilogue(pf_ref, w_refs, b_ref, o_ref):
    p = pf_ref[...]
    z = jnp.dot(p, w_refs[0][...], preferred_element_type=jnp.float32)
    for wr in w_refs[1:]:
        z = jnp.maximum(z, jnp.dot(p, wr[...],
                                   preferred_element_type=jnp.float32))
    o_ref[...] = jnp.maximum(z + b_ref[...], 0.0).astype(o_ref.dtype)


_PHASE1 = {(ph, r): ((2 * ph + r - 1) % 4, (2 * ph + r - 1 - (2 * ph + r - 1) % 4) // 4)
           for ph in range(2) for r in range(4)}
_PHASE2 = {r: ((r - 1) % 2, (r - 1 - (r - 1) % 2) // 2) for r in range(4)}


def _conv1_body(x_ref, w0, w1, w2, w3, b_ref, o_ref, pf_ref):
    IB = x_ref.shape[0]
    pf_ref[...] = jnp.zeros_like(pf_ref)
    for img in range(IB):
        planes = {}
        for ci in range(3):
            pb = x_ref[img, ci].astype(jnp.bfloat16)
            pbm = jnp.concatenate(
                [pb, jnp.zeros((224, 32), jnp.bfloat16)], axis=1)
            pbm = pbm.reshape(56, 1024)
            for pp in range(4):
                planes[(ci, pp)] = pbm[:, pp * 256:pp * 256 + 224]
        for g in range(14):
            cs = 16 * g - 1
            src_lo, src_hi = max(cs, 0), min(cs + 18, 224)
            dst_lo = src_lo - cs
            for ph in range(2):
                for r in range(4):
                    pp, dlt = _PHASE1[(ph, r)]
                    if dlt == 0:
                        qs, qe, ds, de = 0, 56, 0, 56
                    elif dlt < 0:
                        qs, qe, ds, de = 0, 55, 1, 56
                    else:
                        qs, qe, ds, de = 1, 56, 0, 55
                    rb = img * 1568 + g * 112 + ph * 56
                    for ci in range(3):
                        lane = (r * 3 + ci) * 18 + dst_lo
                        pf_ref[rb + ds:rb + de, lane:lane + src_hi - src_lo] = \
                            planes[(ci, pp)][qs:qe, src_lo:src_hi]
    _tap_dots_epilogue(pf_ref, (w0, w1, w2, w3), b_ref, o_ref)


def _conv2_body(y_ref, w0, w1, w2, w3, b_ref, o_ref, pf_ref):
    IB = y_ref.shape[0] // 1568
    pf_ref[...] = jnp.zeros_like(pf_ref)
    for img in range(IB):
        for g2 in range(14):
            for r in range(4):
                ph, dlt = _PHASE2[r]
                if dlt == 0:
                    qs, qe, ds, de = 0, 56, 0, 56
                elif dlt < 0:
                    qs, qe, ds, de = 0, 55, 1, 56
                else:
                    qs, qe, ds, de = 1, 56, 0, 55
                rb = img * 784 + g2 * 56
                lane0 = r * 160
                pieces = []
                if g2 > 0:
                    pieces.append((g2 - 1, 112, 16, lane0))
                pieces.append((g2, 0, 128, lane0 + 16))
                if g2 < 13:
                    pieces.append((g2 + 1, 0, 16, lane0 + 144))
                for (gs, ls, lw, dl) in pieces:
                    sb = img * 1568 + gs * 112 + ph * 56
                    pf_ref[rb + ds:rb + de, dl:dl + lw] = \
                        y_ref[sb + qs:sb + qe, ls:ls + lw]
    _tap_dots_epilogue(pf_ref, (w0, w1, w2, w3), b_ref, o_ref)


def _conv_stage(body, x, xblk, wmats, bias, S, Cout, mrows, ib, kp):
    n_img = x.shape[0] if body is _conv1_body else x.shape[0] // 1568
    grid = (n_img + ib - 1) // ib
    btile = jnp.tile(bias, S).reshape(1, S * Cout).astype(jnp.float32)
    return pl.pallas_call(
        body,
        out_shape=jax.ShapeDtypeStruct((n_img * mrows, S * Cout), jnp.bfloat16),
        grid=(grid,),
        in_specs=[pl.BlockSpec(xblk, lambda i: (i,) + (0,) * (len(xblk) - 1))] + [
            pl.BlockSpec((kp, S * Cout), lambda i: (0, 0))] * 4 + [
            pl.BlockSpec((1, S * Cout), lambda i: (0, 0)),
        ],
        out_specs=pl.BlockSpec((ib * mrows, S * Cout), lambda i: (i, 0)),
        scratch_shapes=[pltpu.VMEM((ib * mrows, kp), jnp.bfloat16)],
        compiler_params=pltpu.CompilerParams(
            dimension_semantics=("parallel",)),
    )(x, *wmats, btile)


def _mlp_body(x_ref, w1_ref, b1_ref, w2_ref, b2_ref, o_ref, acc_ref):
    k = pl.program_id(0)

    @pl.when(k == 0)
    def _():
        acc_ref[...] = jnp.zeros_like(acc_ref)

    w1t = w1_ref[...].reshape(-1, w1_ref.shape[-1])
    acc_ref[...] += jnp.dot(x_ref[...].astype(jnp.float32), w1t,
                            preferred_element_type=jnp.float32)

    @pl.when(k == pl.num_programs(0) - 1)
    def _():
        h = jnp.maximum(acc_ref[...] + b1_ref[...], 0.0)
        o_ref[...] = (jnp.dot(h, w2_ref[...],
                              preferred_element_type=jnp.float32)
                      + b2_ref[...])


def _mlp_head(x, w1v, b1, w2, b2):
    N = x.shape[0]
    Hd = w1v.shape[-1]
    O = w2.shape[1]
    tk = 56 * 4 * 32
    return pl.pallas_call(
        _mlp_body,
        out_shape=jax.ShapeDtypeStruct((N, O), jnp.float32),
        grid=(14,),
        in_specs=[
            pl.BlockSpec((N, tk), lambda k: (0, k)),
            pl.BlockSpec((56, 1, 4, 32, Hd), lambda k: (0, k, 0, 0, 0)),
            pl.BlockSpec((1, Hd), lambda k: (0, 0)),
            pl.BlockSpec((Hd, O), lambda k: (0, 0)),
            pl.BlockSpec((1, O), lambda k: (0, 0)),
        ],
        out_specs=pl.BlockSpec((N, O), lambda k: (0, 0)),
        scratch_shapes=[pltpu.VMEM((N, Hd), jnp.float32)],
        compiler_params=pltpu.CompilerParams(
            dimension_semantics=("arbitrary",),
            vmem_limit_bytes=64 * 1024 * 1024,
        ),
    )(x, w1v, b1.reshape(1, Hd), w2, b2.reshape(1, O))


def kernel(x_nchw, conv1_w, conv1_b, conv2_w, conv2_b, fc1_w, fc1_b,
           fc2_w, fc2_b):
    N = x_nchw.shape[0]
    w1mats = _tap_weights(conv1_w, S=8, Cin=3, Cout=16, ci_major=True)
    w2mats = _tap_weights(conv2_w, S=4, Cin=16, Cout=32, ci_major=False)

    y1 = _conv_stage(_conv1_body, x_nchw, (2, 3, 224, 224), w1mats, conv1_b,
                     S=8, Cout=16, mrows=1568, ib=2, kp=216)
    y2 = _conv_stage(_conv2_body, y1, (2 * 1568, 128), w2mats, conv2_b,
                     S=4, Cout=32, mrows=784, ib=2, kp=640)

    flat = y2.reshape(N, 56 * 56 * 32)
    w1v = fc1_w.reshape(56, 14, 4, 32, 128)
    return _mlp_head(flat, w1v, fc1_b, fc2_w, fc2_b)


# speedup vs baseline: 126.1238x; 1.0145x over previous
import numpy as np
import jax
import jax.numpy as jnp
from jax.experimental import pallas as pl
from jax.experimental.pallas import tpu as pltpu


def _tap_weights(w_mat, S, Cin, Cout, ci_major):
    W = 2 * S + 2
    Kp = 4 * W * Cin
    P = np.zeros((4, Kp, S * 9 * Cin), np.float32)
    for t, (dh, dw) in enumerate([(0, 0), (0, 1), (1, 0), (1, 1)]):
        for s in range(S):
            for kh in range(3):
                for kw in range(3):
                    for ci in range(Cin):
                        r = dh + kh
                        c = 2 * s + dw + kw
                        if ci_major:
                            dst = (r * Cin + ci) * W + c
                        else:
                            dst = (r * W + c) * Cin + ci
                        src = (kh * 3 + kw) * Cin + ci
                        P[t, dst, s * 9 * Cin + src] = 1.0
    wrep = (jnp.eye(S, dtype=w_mat.dtype)[:, None, :, None]
            * w_mat[None, :, None, :]).reshape(S * 9 * Cin, S * Cout)
    wall = jnp.einsum('tkm,mn->tkn', jnp.asarray(P), wrep).astype(jnp.bfloat16)
    return [wall[t] for t in range(4)]


def _tap_dots_epilogue(pf_ref, w_refs, b_ref, o_ref):
    p = pf_ref[...]
    z = jnp.dot(p, w_refs[0][...], preferred_element_type=jnp.float32)
    for wr in w_refs[1:]:
        z = jnp.maximum(z, jnp.dot(p, wr[...],
                                   preferred_element_type=jnp.float32))
    o_ref[...] = jnp.maximum(z + b_ref[...], 0.0).astype(o_ref.dtype)


_PHASE1 = {(ph, r): ((2 * ph + r - 1) % 4, (2 * ph + r - 1 - (2 * ph + r - 1) % 4) // 4)
           for ph in range(2) for r in range(4)}
_PHASE2 = {r: ((r - 1) % 2, (r - 1 - (r - 1) % 2) // 2) for r in range(4)}


def _conv1_body(x_ref, w0, w1, w2, w3, b_ref, o_ref, pf_ref):
    IB = x_ref.shape[0]
    pf_ref[...] = jnp.zeros_like(pf_ref)
    for img in range(IB):
        planes = {}
        for ci in range(3):
            pb = x_ref[img, ci].astype(jnp.bfloat16)
            pbm = jnp.concatenate(
                [pb, jnp.zeros((224, 32), jnp.bfloat16)], axis=1)
            pbm = pbm.reshape(56, 1024)
            for pp in range(4):
                planes[(ci, pp)] = pbm[:, pp * 256:pp * 256 + 224]
        for g in range(14):
            cs = 16 * g - 1
            src_lo, src_hi = max(cs, 0), min(cs + 18, 224)
            dst_lo = src_lo - cs
            for ph in range(2):
                for r in range(4):
                    pp, dlt = _PHASE1[(ph, r)]
                    if dlt == 0:
                        qs, qe, ds, de = 0, 56, 0, 56
                    elif dlt < 0:
                        qs, qe, ds, de = 0, 55, 1, 56
                    else:
                        qs, qe, ds, de = 1, 56, 0, 55
                    rb = img * 1568 + g * 112 + ph * 56
                    for ci in range(3):
                        lane = (r * 3 + ci) * 18 + dst_lo
                        pf_ref[rb + ds:rb + de, lane:lane + src_hi - src_lo] = \
                            planes[(ci, pp)][qs:qe, src_lo:src_hi]
    _tap_dots_epilogue(pf_ref, (w0, w1, w2, w3), b_ref, o_ref)


def _conv2_body(y_ref, w0, w1, w2, w3, b_ref, o_ref, pf_ref):
    IB = y_ref.shape[0] // 1568
    pf_ref[...] = jnp.zeros_like(pf_ref)
    for img in range(IB):
        for g2 in range(14):
            for r in range(4):
                ph, dlt = _PHASE2[r]
                if dlt == 0:
                    qs, qe, ds, de = 0, 56, 0, 56
                elif dlt < 0:
                    qs, qe, ds, de = 0, 55, 1, 56
                else:
                    qs, qe, ds, de = 1, 56, 0, 55
                rb = img * 784 + g2 * 56
                lane0 = r * 160
                pieces = []
                if g2 > 0:
                    pieces.append((g2 - 1, 112, 16, lane0))
                pieces.append((g2, 0, 128, lane0 + 16))
                if g2 < 13:
                    pieces.append((g2 + 1, 0, 16, lane0 + 144))
                for (gs, ls, lw, dl) in pieces:
                    sb = img * 1568 + gs * 112 + ph * 56
                    pf_ref[rb + ds:rb + de, dl:dl + lw] = \
                        y_ref[sb + qs:sb + qe, ls:ls + lw]
    _tap_dots_epilogue(pf_ref, (w0, w1, w2, w3), b_ref, o_ref)


def _conv_stage(body, x, xblk, wmats, bias, S, Cout, mrows, ib, kp):
    n_img = x.shape[0] if body is _conv1_body else x.shape[0] // 1568
    grid = (n_img + ib - 1) // ib
    btile = jnp.tile(bias, S).reshape(1, S * Cout).astype(jnp.float32)
    return pl.pallas_call(
        body,
        out_shape=jax.ShapeDtypeStruct((n_img * mrows, S * Cout), jnp.bfloat16),
        grid=(grid,),
        in_specs=[pl.BlockSpec(xblk, lambda i: (i,) + (0,) * (len(xblk) - 1))] + [
            pl.BlockSpec((kp, S * Cout), lambda i: (0, 0))] * 4 + [
            pl.BlockSpec((1, S * Cout), lambda i: (0, 0)),
        ],
        out_specs=pl.BlockSpec((ib * mrows, S * Cout), lambda i: (i, 0)),
        scratch_shapes=[pltpu.VMEM((ib * mrows, kp), jnp.bfloat16)],
        compiler_params=pltpu.CompilerParams(
            dimension_semantics=("parallel",)),
    )(x, *wmats, btile)


def _mlp_body(x_ref, w1_ref, b1_ref, w2_ref, b2_ref, o_ref, acc_ref):
    k = pl.program_id(0)

    @pl.when(k == 0)
    def _():
        acc_ref[...] = jnp.zeros_like(acc_ref)

    w1t = w1_ref[...].reshape(-1, w1_ref.shape[-1])
    acc_ref[...] += jnp.dot(x_ref[...].astype(jnp.float32), w1t,
                            preferred_element_type=jnp.float32)

    @pl.when(k == pl.num_programs(0) - 1)
    def _():
        h = jnp.maximum(acc_ref[...] + b1_ref[...], 0.0)
        o_ref[...] = (jnp.dot(h, w2_ref[...],
                              preferred_element_type=jnp.float32)
                      + b2_ref[...])


def _mlp_head(x, w1v, b1, w2, b2):
    N = x.shape[0]
    Hd = w1v.shape[-1]
    O = w2.shape[1]
    tk = 56 * 4 * 32
    return pl.pallas_call(
        _mlp_body,
        out_shape=jax.ShapeDtypeStruct((N, O), jnp.float32),
        grid=(14,),
        in_specs=[
            pl.BlockSpec((N, tk), lambda k: (0, k)),
            pl.BlockSpec((56, 1, 4, 32, Hd), lambda k: (0, k, 0, 0, 0)),
            pl.BlockSpec((1, Hd), lambda k: (0, 0)),
            pl.BlockSpec((Hd, O), lambda k: (0, 0)),
            pl.BlockSpec((1, O), lambda k: (0, 0)),
        ],
        out_specs=pl.BlockSpec((N, O), lambda k: (0, 0)),
        scratch_shapes=[pltpu.VMEM((N, Hd), jnp.float32)],
        compiler_params=pltpu.CompilerParams(
            dimension_semantics=("arbitrary",),
            vmem_limit_bytes=64 * 1024 * 1024,
        ),
    )(x, w1v, b1.reshape(1, Hd), w2, b2.reshape(1, O))


def kernel(x_nchw, conv1_w, conv1_b, conv2_w, conv2_b, fc1_w, fc1_b,
           fc2_w, fc2_b):
    N = x_nchw.shape[0]
    w1mats = _tap_weights(conv1_w, S=8, Cin=3, Cout=16, ci_major=True)
    w2mats = _tap_weights(conv2_w, S=4, Cin=16, Cout=32, ci_major=False)

    y1 = _conv_stage(_conv1_body, x_nchw, (4, 3, 224, 224), w1mats, conv1_b,
                     S=8, Cout=16, mrows=1568, ib=4, kp=216)
    y2 = _conv_stage(_conv2_body, y1, (4 * 1568, 128), w2mats, conv2_b,
                     S=4, Cout=32, mrows=784, ib=4, kp=640)

    flat = y2.reshape(N, 56 * 56 * 32)
    w1v = fc1_w.reshape(56, 14, 4, 32, 128)
    return _mlp_head(flat, w1v, fc1_b, fc2_w, fc2_b)


# speedup vs baseline: 134.7424x; 1.0683x over previous
import numpy as np
import jax
import jax.numpy as jnp
from jax.experimental import pallas as pl
from jax.experimental.pallas import tpu as pltpu


def _tap_weights(w_mat, S, Cin, Cout, ci_major):
    W = 2 * S + 2
    Kp = 4 * W * Cin
    P = np.zeros((4, Kp, S * 9 * Cin), np.float32)
    for t, (dh, dw) in enumerate([(0, 0), (0, 1), (1, 0), (1, 1)]):
        for s in range(S):
            for kh in range(3):
                for kw in range(3):
                    for ci in range(Cin):
                        r = dh + kh
                        c = 2 * s + dw + kw
                        if ci_major:
                            dst = (r * Cin + ci) * W + c
                        else:
                            dst = (r * W + c) * Cin + ci
                        src = (kh * 3 + kw) * Cin + ci
                        P[t, dst, s * 9 * Cin + src] = 1.0
    wrep = (jnp.eye(S, dtype=w_mat.dtype)[:, None, :, None]
            * w_mat[None, :, None, :]).reshape(S * 9 * Cin, S * Cout)
    wall = jnp.einsum('tkm,mn->tkn', jnp.asarray(P), wrep).astype(jnp.bfloat16)
    return [wall[t] for t in range(4)]


def _tap_dots_epilogue(pf_ref, w_refs, b_ref, o_ref):
    p = pf_ref[...]
    z = jnp.dot(p, w_refs[0][...], preferred_element_type=jnp.float32)
    for wr in w_refs[1:]:
        z = jnp.maximum(z, jnp.dot(p, wr[...],
                                   preferred_element_type=jnp.float32))
    o_ref[...] = jnp.maximum(z + b_ref[...], 0.0).astype(o_ref.dtype)


_PHASE1 = {(ph, r): ((2 * ph + r - 1) % 4, (2 * ph + r - 1 - (2 * ph + r - 1) % 4) // 4)
           for ph in range(2) for r in range(4)}
_PHASE2 = {r: ((r - 1) % 2, (r - 1 - (r - 1) % 2) // 2) for r in range(4)}


def _conv1_patches(x_ref, pf_ref):
    IB = x_ref.shape[0]
    pf_ref[...] = jnp.zeros_like(pf_ref)
    for img in range(IB):
        planes = {}
        for ci in range(3):
            pb = x_ref[img, ci].astype(jnp.bfloat16)
            pbm = jnp.concatenate(
                [pb, jnp.zeros((224, 32), jnp.bfloat16)], axis=1)
            pbm = pbm.reshape(56, 1024)
            for pp in range(4):
                planes[(ci, pp)] = pbm[:, pp * 256:pp * 256 + 224]
        for g in range(14):
            cs = 16 * g - 1
            src_lo, src_hi = max(cs, 0), min(cs + 18, 224)
            dst_lo = src_lo - cs
            for ph in range(2):
                for r in range(4):
                    pp, dlt = _PHASE1[(ph, r)]
                    if dlt == 0:
                        qs, qe, ds, de = 0, 56, 0, 56
                    elif dlt < 0:
                        qs, qe, ds, de = 0, 55, 1, 56
                    else:
                        qs, qe, ds, de = 1, 56, 0, 55
                    rb = img * 1568 + g * 112 + ph * 56
                    for ci in range(3):
                        lane = (r * 3 + ci) * 18 + dst_lo
                        pf_ref[rb + ds:rb + de, lane:lane + src_hi - src_lo] = \
                            planes[(ci, pp)][qs:qe, src_lo:src_hi]


def _conv2_patches(y_ref, pf_ref):
    IB = y_ref.shape[0] // 1568
    pf_ref[...] = jnp.zeros_like(pf_ref)
    for img in range(IB):
        for g2 in range(14):
            for r in range(4):
                ph, dlt = _PHASE2[r]
                if dlt == 0:
                    qs, qe, ds, de = 0, 56, 0, 56
                elif dlt < 0:
                    qs, qe, ds, de = 0, 55, 1, 56
                else:
                    qs, qe, ds, de = 1, 56, 0, 55
                rb = img * 784 + g2 * 56
                lane0 = r * 160
                pieces = []
                if g2 > 0:
                    pieces.append((g2 - 1, 112, 16, lane0))
                pieces.append((g2, 0, 128, lane0 + 16))
                if g2 < 13:
                    pieces.append((g2 + 1, 0, 16, lane0 + 144))
                for (gs, ls, lw, dl) in pieces:
                    sb = img * 1568 + gs * 112 + ph * 56
                    pf_ref[rb + ds:rb + de, dl:dl + lw] = \
                        y_ref[sb + qs:sb + qe, ls:ls + lw]


def _convs_body(x_ref, cw0, cw1, cw2, cw3, b1_ref, dw0, dw1, dw2, dw3, b2_ref,
                o_ref, pf1_ref, y1_ref, pf2_ref):
    _conv1_patches(x_ref, pf1_ref)
    _tap_dots_epilogue(pf1_ref, (cw0, cw1, cw2, cw3), b1_ref, y1_ref)
    _conv2_patches(y1_ref, pf2_ref)
    _tap_dots_epilogue(pf2_ref, (dw0, dw1, dw2, dw3), b2_ref, o_ref)


def _conv_stages(x, w1mats, b1, w2mats, b2, ib):
    n_img = x.shape[0]
    bt1 = jnp.tile(b1, 8).reshape(1, 128).astype(jnp.float32)
    bt2 = jnp.tile(b2, 4).reshape(1, 128).astype(jnp.float32)
    return pl.pallas_call(
        _convs_body,
        out_shape=jax.ShapeDtypeStruct((n_img * 784, 128), jnp.bfloat16),
        grid=(n_img // ib,),
        in_specs=[pl.BlockSpec((ib, 3, 224, 224), lambda i: (i, 0, 0, 0))] + [
            pl.BlockSpec((216, 128), lambda i: (0, 0))] * 4 + [
            pl.BlockSpec((1, 128), lambda i: (0, 0))] + [
            pl.BlockSpec((640, 128), lambda i: (0, 0))] * 4 + [
            pl.BlockSpec((1, 128), lambda i: (0, 0)),
        ],
        out_specs=pl.BlockSpec((ib * 784, 128), lambda i: (i, 0)),
        scratch_shapes=[
            pltpu.VMEM((ib * 1568, 216), jnp.bfloat16),
            pltpu.VMEM((ib * 1568, 128), jnp.bfloat16),
            pltpu.VMEM((ib * 784, 640), jnp.bfloat16),
        ],
        compiler_params=pltpu.CompilerParams(
            dimension_semantics=("parallel",)),
    )(x, *w1mats, bt1, *w2mats, bt2)


def _mlp_body(x_ref, w1_ref, b1_ref, w2_ref, b2_ref, o_ref, acc_ref):
    k = pl.program_id(0)

    @pl.when(k == 0)
    def _():
        acc_ref[...] = jnp.zeros_like(acc_ref)

    w1t = w1_ref[...].reshape(-1, w1_ref.shape[-1])
    acc_ref[...] += jnp.dot(x_ref[...].astype(jnp.float32), w1t,
                            preferred_element_type=jnp.float32)

    @pl.when(k == pl.num_programs(0) - 1)
    def _():
        h = jnp.maximum(acc_ref[...] + b1_ref[...], 0.0)
        o_ref[...] = (jnp.dot(h, w2_ref[...],
                              preferred_element_type=jnp.float32)
                      + b2_ref[...])


def _mlp_head(x, w1v, b1, w2, b2):
    N = x.shape[0]
    Hd = w1v.shape[-1]
    O = w2.shape[1]
    tk = 56 * 4 * 32
    return pl.pallas_call(
        _mlp_body,
        out_shape=jax.ShapeDtypeStruct((N, O), jnp.float32),
        grid=(14,),
        in_specs=[
            pl.BlockSpec((N, tk), lambda k: (0, k)),
            pl.BlockSpec((56, 1, 4, 32, Hd), lambda k: (0, k, 0, 0, 0)),
            pl.BlockSpec((1, Hd), lambda k: (0, 0)),
            pl.BlockSpec((Hd, O), lambda k: (0, 0)),
            pl.BlockSpec((1, O), lambda k: (0, 0)),
        ],
        out_specs=pl.BlockSpec((N, O), lambda k: (0, 0)),
        scratch_shapes=[pltpu.VMEM((N, Hd), jnp.float32)],
        compiler_params=pltpu.CompilerParams(
            dimension_semantics=("arbitrary",),
            vmem_limit_bytes=64 * 1024 * 1024,
        ),
    )(x, w1v, b1.reshape(1, Hd), w2, b2.reshape(1, O))


def kernel(x_nchw, conv1_w, conv1_b, conv2_w, conv2_b, fc1_w, fc1_b,
           fc2_w, fc2_b):
    N = x_nchw.shape[0]
    w1mats = _tap_weights(conv1_w, S=8, Cin=3, Cout=16, ci_major=True)
    w2mats = _tap_weights(conv2_w, S=4, Cin=16, Cout=32, ci_major=False)

    y2 = _conv_stages(x_nchw, w1mats, conv1_b, w2mats, conv2_b, ib=4)

    flat = y2.reshape(N, 56 * 56 * 32)
    w1v = fc1_w.reshape(56, 14, 4, 32, 128)
    return _mlp_head(flat, w1v, fc1_b, fc2_w, fc2_b)


# speedup vs baseline: 155.4179x; 1.1534x over previous
import numpy as np
import jax
import jax.numpy as jnp
from jax.experimental import pallas as pl
from jax.experimental.pallas import tpu as pltpu


def _tap_weights(w_mat, S, Cin, Cout, ci_major):
    W = 2 * S + 2
    Kp = 4 * W * Cin
    P = np.zeros((4, Kp, S * 9 * Cin), np.float32)
    for t, (dh, dw) in enumerate([(0, 0), (0, 1), (1, 0), (1, 1)]):
        for s in range(S):
            for kh in range(3):
                for kw in range(3):
                    for ci in range(Cin):
                        r = dh + kh
                        c = 2 * s + dw + kw
                        if ci_major:
                            dst = (r * Cin + ci) * W + c
                        else:
                            dst = (r * W + c) * Cin + ci
                        src = (kh * 3 + kw) * Cin + ci
                        P[t, dst, s * 9 * Cin + src] = 1.0
    wrep = (jnp.eye(S, dtype=w_mat.dtype)[:, None, :, None]
            * w_mat[None, :, None, :]).reshape(S * 9 * Cin, S * Cout)
    wall = jnp.einsum('tkm,mn->tkn', jnp.asarray(P), wrep).astype(jnp.bfloat16)
    return [jnp.concatenate([wall[0], wall[1]], axis=1),
            jnp.concatenate([wall[2], wall[3]], axis=1)]


def _tap_dots_epilogue(pf_ref, wa_ref, wb_ref, b_ref, o_ref):
    p = pf_ref[...]
    za = jnp.dot(p, wa_ref[...], preferred_element_type=jnp.float32)
    zb = jnp.dot(p, wb_ref[...], preferred_element_type=jnp.float32)
    z = jnp.maximum(za, zb)
    z = jnp.maximum(z[:, :128], z[:, 128:])
    o_ref[...] = jnp.maximum(z + b_ref[...], 0.0).astype(o_ref.dtype)


_PHASE1 = {(ph, r): ((2 * ph + r - 1) % 4, (2 * ph + r - 1 - (2 * ph + r - 1) % 4) // 4)
           for ph in range(2) for r in range(4)}
_PHASE2 = {r: ((r - 1) % 2, (r - 1 - (r - 1) % 2) // 2) for r in range(4)}


def _conv1_patches(x_ref, pf_ref):
    IB = x_ref.shape[0]
    pf_ref[...] = jnp.zeros_like(pf_ref)
    for img in range(IB):
        planes = {}
        for ci in range(3):
            pb = x_ref[img, ci].astype(jnp.bfloat16)
            pbm = jnp.concatenate(
                [pb, jnp.zeros((224, 32), jnp.bfloat16)], axis=1)
            pbm = pbm.reshape(56, 1024)
            for pp in range(4):
                planes[(ci, pp)] = pbm[:, pp * 256:pp * 256 + 224]
        for g in range(14):
            cs = 16 * g - 1
            src_lo, src_hi = max(cs, 0), min(cs + 18, 224)
            dst_lo = src_lo - cs
            for ph in range(2):
                for r in range(4):
                    pp, dlt = _PHASE1[(ph, r)]
                    if dlt == 0:
                        qs, qe, ds, de = 0, 56, 0, 56
                    elif dlt < 0:
                        qs, qe, ds, de = 0, 55, 1, 56
                    else:
                        qs, qe, ds, de = 1, 56, 0, 55
                    rb = img * 1568 + g * 112 + ph * 56
                    for ci in range(3):
                        lane = (r * 3 + ci) * 18 + dst_lo
                        pf_ref[rb + ds:rb + de, lane:lane + src_hi - src_lo] = \
                            planes[(ci, pp)][qs:qe, src_lo:src_hi]


def _conv2_patches(y_ref, pf_ref):
    IB = y_ref.shape[0] // 1568
    pf_ref[...] = jnp.zeros_like(pf_ref)
    for img in range(IB):
        for g2 in range(14):
            for r in range(4):
                ph, dlt = _PHASE2[r]
                if dlt == 0:
                    qs, qe, ds, de = 0, 56, 0, 56
                elif dlt < 0:
                    qs, qe, ds, de = 0, 55, 1, 56
                else:
                    qs, qe, ds, de = 1, 56, 0, 55
                rb = img * 784 + g2 * 56
                lane0 = r * 160
                pieces = []
                if g2 > 0:
                    pieces.append((g2 - 1, 112, 16, lane0))
                pieces.append((g2, 0, 128, lane0 + 16))
                if g2 < 13:
                    pieces.append((g2 + 1, 0, 16, lane0 + 144))
                for (gs, ls, lw, dl) in pieces:
                    sb = img * 1568 + gs * 112 + ph * 56
                    pf_ref[rb + ds:rb + de, dl:dl + lw] = \
                        y_ref[sb + qs:sb + qe, ls:ls + lw]


def _convs_body(x_ref, cwa, cwb, b1_ref, dwa, dwb, b2_ref,
                o_ref, pf1_ref, y1_ref, pf2_ref):
    _conv1_patches(x_ref, pf1_ref)
    _tap_dots_epilogue(pf1_ref, cwa, cwb, b1_ref, y1_ref)
    _conv2_patches(y1_ref, pf2_ref)
    _tap_dots_epilogue(pf2_ref, dwa, dwb, b2_ref, o_ref)


def _conv_stages(x, w1mats, b1, w2mats, b2, ib):
    n_img = x.shape[0]
    bt1 = jnp.tile(b1, 8).reshape(1, 128).astype(jnp.float32)
    bt2 = jnp.tile(b2, 4).reshape(1, 128).astype(jnp.float32)
    return pl.pallas_call(
        _convs_body,
        out_shape=jax.ShapeDtypeStruct((n_img * 784, 128), jnp.bfloat16),
        grid=(n_img // ib,),
        in_specs=[pl.BlockSpec((ib, 3, 224, 224), lambda i: (i, 0, 0, 0))] + [
            pl.BlockSpec((216, 256), lambda i: (0, 0))] * 2 + [
            pl.BlockSpec((1, 128), lambda i: (0, 0))] + [
            pl.BlockSpec((640, 256), lambda i: (0, 0))] * 2 + [
            pl.BlockSpec((1, 128), lambda i: (0, 0)),
        ],
        out_specs=pl.BlockSpec((ib * 784, 128), lambda i: (i, 0)),
        scratch_shapes=[
            pltpu.VMEM((ib * 1568, 216), jnp.bfloat16),
            pltpu.VMEM((ib * 1568, 128), jnp.bfloat16),
            pltpu.VMEM((ib * 784, 640), jnp.bfloat16),
        ],
        compiler_params=pltpu.CompilerParams(
            dimension_semantics=("parallel",)),
    )(x, *w1mats, bt1, *w2mats, bt2)


def _mlp_body(x_ref, w1_ref, b1_ref, w2_ref, b2_ref, o_ref, acc_ref):
    k = pl.program_id(0)

    @pl.when(k == 0)
    def _():
        acc_ref[...] = jnp.zeros_like(acc_ref)

    w1t = w1_ref[...].reshape(-1, w1_ref.shape[-1])
    acc_ref[...] += jnp.dot(x_ref[...].astype(jnp.float32), w1t,
                            preferred_element_type=jnp.float32)

    @pl.when(k == pl.num_programs(0) - 1)
    def _():
        h = jnp.maximum(acc_ref[...] + b1_ref[...], 0.0)
        o_ref[...] = (jnp.dot(h, w2_ref[...],
                              preferred_element_type=jnp.float32)
                      + b2_ref[...])


def _mlp_head(x, w1v, b1, w2, b2):
    N = x.shape[0]
    Hd = w1v.shape[-1]
    O = w2.shape[1]
    tk = 56 * 4 * 32
    return pl.pallas_call(
        _mlp_body,
        out_shape=jax.ShapeDtypeStruct((N, O), jnp.float32),
        grid=(14,),
        in_specs=[
            pl.BlockSpec((N, tk), lambda k: (0, k)),
            pl.BlockSpec((56, 1, 4, 32, Hd), lambda k: (0, k, 0, 0, 0)),
            pl.BlockSpec((1, Hd), lambda k: (0, 0)),
            pl.BlockSpec((Hd, O), lambda k: (0, 0)),
            pl.BlockSpec((1, O), lambda k: (0, 0)),
        ],
        out_specs=pl.BlockSpec((N, O), lambda k: (0, 0)),
        scratch_shapes=[pltpu.VMEM((N, Hd), jnp.float32)],
        compiler_params=pltpu.CompilerParams(
            dimension_semantics=("arbitrary",),
            vmem_limit_bytes=64 * 1024 * 1024,
        ),
    )(x, w1v, b1.reshape(1, Hd), w2, b2.reshape(1, O))


def kernel(x_nchw, conv1_w, conv1_b, conv2_w, conv2_b, fc1_w, fc1_b,
           fc2_w, fc2_b):
    N = x_nchw.shape[0]
    w1mats = _tap_weights(conv1_w, S=8, Cin=3, Cout=16, ci_major=True)
    w2mats = _tap_weights(conv2_w, S=4, Cin=16, Cout=32, ci_major=False)

    y2 = _conv_stages(x_nchw, w1mats, conv1_b, w2mats, conv2_b, ib=4)

    flat = y2.reshape(N, 56 * 56 * 32)
    w1v = fc1_w.reshape(56, 14, 4, 32, 128)
    return _mlp_head(flat, w1v, fc1_b, fc2_w, fc2_b)
